# baseline (speedup 1.0000x reference)
import functools

import jax
import jax.numpy as jnp
from jax import lax
from jax.experimental import pallas as pl
from jax.experimental.pallas import tpu as pltpu
from jax.experimental.pallas import tpu_sc as plsc

MAX_DIST_K = 100
HEADS = 8
LANES = 16
NUM_WORKERS = 32
ROWS_PER_CHUNK = 16
SUBLANES = 8
LANE_TILE = 128
TABLE_ROWS = MAX_DIST_K + 2


def _sc_lookup(dist_t, table_rep, b, n):
    batches_per_w = b // NUM_WORKERS
    chunks_per_b = n // ROWS_PER_CHUNK
    ntiles = n // LANE_TILE
    rowgrp = ROWS_PER_CHUNK // SUBLANES
    mesh = plsc.VectorSubcoreMesh(core_axis_name="c", subcore_axis_name="s")

    @functools.partial(
        pl.kernel,
        mesh=mesh,
        compiler_params=pltpu.CompilerParams(
            needs_layout_passes=False, use_tc_tiling_on_sc=False),
        out_type=jax.ShapeDtypeStruct((b, n, ntiles, HEADS, LANE_TILE),
                                      jnp.float32),
        scratch_types=[
            pltpu.VMEM((TABLE_ROWS * HEADS * LANES,), jnp.float32),
            pltpu.VMEM((2, rowgrp, ntiles, SUBLANES, LANE_TILE), jnp.int32),
            pltpu.VMEM((2, ROWS_PER_CHUNK, ntiles, HEADS, LANE_TILE),
                       jnp.float32),
            pltpu.SemaphoreType.DMA,
            pltpu.SemaphoreType.DMA,
            pltpu.SemaphoreType.DMA,
            pltpu.SemaphoreType.DMA,
        ],
    )
    def k(dist_hbm, table_hbm, out_hbm, table_v, din, out_v,
          sin0, sin1, sout0, sout1):
        wid = lax.axis_index("s") * 2 + lax.axis_index("c")
        pltpu.sync_copy(table_hbm, table_v)
        lane = lax.iota(jnp.int32, LANES)
        nblk = LANE_TILE // LANES
        nchunks = batches_per_w * chunks_per_b
        sins = (sin0, sin1)
        souts = (sout0, sout1)

        def din_src(g):
            bb = wid * batches_per_w + g // chunks_per_b
            return dist_hbm.at[bb, pl.ds((g % chunks_per_b) * rowgrp, rowgrp)]

        def out_dst(g):
            bb = wid * batches_per_w + g // chunks_per_b
            r0 = (g % chunks_per_b) * ROWS_PER_CHUNK
            return out_hbm.at[bb, pl.ds(r0, ROWS_PER_CHUNK)]

        for bslot in range(2):
            pltpu.make_async_copy(din_src(bslot), din.at[bslot],
                                  sins[bslot]).start()

        def pair_body(g0, carry):
            for bslot in range(2):
                g = g0 + bslot
                dslot, oslot = din.at[bslot], out_v.at[bslot]
                pltpu.make_async_copy(din_src(g), dslot, sins[bslot]).wait()

                @pl.when(g >= 2)
                def _():
                    pltpu.make_async_copy(oslot, out_dst(g - 2),
                                          souts[bslot]).wait()

                @plsc.parallel_loop(0, ROWS_PER_CHUNK * n // LANES, unroll=8)
                def vec_body(i):
                    jb = i & (nblk - 1)
                    t = (i >> 3) & (ntiles - 1)
                    s = (i >> 4) & (SUBLANES - 1)
                    p = i >> 7
                    lb = jb * LANES
                    v = dslot[p, t, s, pl.ds(lb, LANES)]
                    a = (jnp.minimum(jnp.maximum(v, -1), MAX_DIST_K) + 1) * (
                        HEADS * LANES) + lane
                    row = p * SUBLANES + s
                    for h in range(HEADS):
                        g_h = plsc.load_gather(table_v, [a + h * LANES])
                        oslot[row, t, h, pl.ds(lb, LANES)] = g_h

                pltpu.make_async_copy(oslot, out_dst(g), souts[bslot]).start()

                @pl.when(g + 2 < nchunks)
                def _():
                    pltpu.make_async_copy(din_src(g + 2), din.at[bslot],
                                          sins[bslot]).start()
            return carry

        lax.fori_loop(0, nchunks // 2, lambda q, c: pair_body(q * 2, c), 0)
        for bslot in range(2):
            pltpu.make_async_copy(out_v.at[bslot],
                                  out_dst(nchunks - 2 + bslot),
                                  souts[bslot]).wait()

    return k(dist_t, table_rep)


def kernel(dist, table):
    b, n, _ = dist.shape
    table_rep = jnp.broadcast_to(
        table.reshape(-1)[:, None], (TABLE_ROWS * HEADS, LANES)).reshape(-1)
    dist_t = dist.reshape(b, n // SUBLANES, SUBLANES, n // LANE_TILE,
                          LANE_TILE).transpose(0, 1, 3, 2, 4)
    y = _sc_lookup(dist_t, table_rep, b, n)
    return jnp.transpose(y, (0, 1, 2, 4, 3)).reshape(b, n, n, HEADS)


# speedup vs baseline: 1.0017x; 1.0017x over previous
import functools

import jax
import jax.numpy as jnp
from jax import lax
from jax.experimental import pallas as pl
from jax.experimental.pallas import tpu as pltpu
from jax.experimental.pallas import tpu_sc as plsc

MAX_DIST_K = 100
HEADS = 8
LANES = 16
NUM_WORKERS = 32
ROWS_PER_CHUNK = 16
SUBLANES = 8
LANE_TILE = 128
TABLE_ROWS = MAX_DIST_K + 2


def _sc_lookup(dist_t, table_rep, b, n):
    batches_per_w = b // NUM_WORKERS
    chunks_per_b = n // ROWS_PER_CHUNK
    ntiles = n // LANE_TILE
    rowgrp = ROWS_PER_CHUNK // SUBLANES
    mesh = plsc.VectorSubcoreMesh(core_axis_name="c", subcore_axis_name="s")

    @functools.partial(
        pl.kernel,
        mesh=mesh,
        compiler_params=pltpu.CompilerParams(
            needs_layout_passes=False, use_tc_tiling_on_sc=False),
        out_type=jax.ShapeDtypeStruct((b, n, ntiles, HEADS, LANE_TILE),
                                      jnp.float32),
        scratch_types=[
            pltpu.VMEM((TABLE_ROWS * HEADS * LANES,), jnp.float32),
            pltpu.VMEM((2, rowgrp, ntiles, SUBLANES, LANE_TILE), jnp.int32),
            pltpu.VMEM((2, ROWS_PER_CHUNK, ntiles, HEADS, LANE_TILE),
                       jnp.float32),
            pltpu.SemaphoreType.DMA,
            pltpu.SemaphoreType.DMA,
            pltpu.SemaphoreType.DMA,
            pltpu.SemaphoreType.DMA,
        ],
    )
    def k(dist_hbm, table_hbm, out_hbm, table_v, din, out_v,
          sin0, sin1, sout0, sout1):
        wid = lax.axis_index("s") * 2 + lax.axis_index("c")
        pltpu.sync_copy(table_hbm, table_v)
        lane = lax.iota(jnp.int32, LANES)
        nblk = LANE_TILE // LANES
        nchunks = batches_per_w * chunks_per_b
        sins = (sin0, sin1)
        souts = (sout0, sout1)

        def din_src(g):
            bb = wid * batches_per_w + g // chunks_per_b
            return dist_hbm.at[bb, pl.ds((g % chunks_per_b) * rowgrp, rowgrp)]

        def out_dst(g):
            bb = wid * batches_per_w + g // chunks_per_b
            r0 = (g % chunks_per_b) * ROWS_PER_CHUNK
            return out_hbm.at[bb, pl.ds(r0, ROWS_PER_CHUNK)]

        for bslot in range(2):
            pltpu.make_async_copy(din_src(bslot), din.at[bslot],
                                  sins[bslot]).start()

        def pair_body(g0, carry):
            for bslot in range(2):
                g = g0 + bslot
                dslot, oslot = din.at[bslot], out_v.at[bslot]
                pltpu.make_async_copy(din_src(g), dslot, sins[bslot]).wait()

                @pl.when(g >= 2)
                def _():
                    pltpu.make_async_copy(oslot, out_dst(g - 2),
                                          souts[bslot]).wait()

                @plsc.parallel_loop(0, ROWS_PER_CHUNK * n // LANES, unroll=4)
                def vec_body(i):
                    jb = i & (nblk - 1)
                    t = (i >> 3) & (ntiles - 1)
                    s = (i >> 4) & (SUBLANES - 1)
                    p = i >> 7
                    lb = jb * LANES
                    v = dslot[p, t, s, pl.ds(lb, LANES)]
                    a = (jnp.minimum(jnp.maximum(v, -1), MAX_DIST_K) + 1) * (
                        HEADS * LANES) + lane
                    row = p * SUBLANES + s
                    for h in range(HEADS):
                        g_h = plsc.load_gather(table_v, [a + h * LANES])
                        oslot[row, t, h, pl.ds(lb, LANES)] = g_h

                pltpu.make_async_copy(oslot, out_dst(g), souts[bslot]).start()

                @pl.when(g + 2 < nchunks)
                def _():
                    pltpu.make_async_copy(din_src(g + 2), din.at[bslot],
                                          sins[bslot]).start()
            return carry

        lax.fori_loop(0, nchunks // 2, lambda q, c: pair_body(q * 2, c), 0)
        for bslot in range(2):
            pltpu.make_async_copy(out_v.at[bslot],
                                  out_dst(nchunks - 2 + bslot),
                                  souts[bslot]).wait()

    return k(dist_t, table_rep)


def kernel(dist, table):
    b, n, _ = dist.shape
    table_rep = jnp.broadcast_to(
        table.reshape(-1)[:, None], (TABLE_ROWS * HEADS, LANES)).reshape(-1)
    dist_t = dist.reshape(b, n // SUBLANES, SUBLANES, n // LANE_TILE,
                          LANE_TILE).transpose(0, 1, 3, 2, 4)
    y = _sc_lookup(dist_t, table_rep, b, n)
    return jnp.transpose(y, (0, 1, 2, 4, 3)).reshape(b, n, n, HEADS)
